# baseline (speedup 1.0000x reference)
import jax
import jax.numpy as jnp
from jax.experimental import pallas as pl

_PREC = jax.lax.Precision.DEFAULT


def _mlp_block(x_ref, w1_ref, b1_ref, w2_ref, b2_ref, out_ref):
    h = jnp.dot(x_ref[:], w1_ref[:], precision=_PREC,
                preferred_element_type=jnp.float32)
    h = jnp.maximum(h + b1_ref[:], 0.0)
    out = jnp.dot(h, w2_ref[:], precision=_PREC,
                  preferred_element_type=jnp.float32)
    out_ref[:] = out + b2_ref[:]


def kernel(x, edge_index, W1, b1, W2, b2):
    n, d = x.shape
    hid = W1.shape[1]
    ncls = W2.shape[1]
    b1r = b1.reshape(1, hid)
    b2r = b2.reshape(1, ncls)
    return pl.pallas_call(
        _mlp_block,
        grid=(1,),
        in_specs=[
            pl.BlockSpec((n, d), lambda i: (0, 0)),
            pl.BlockSpec((d, hid), lambda i: (0, 0)),
            pl.BlockSpec((1, hid), lambda i: (0, 0)),
            pl.BlockSpec((hid, ncls), lambda i: (0, 0)),
            pl.BlockSpec((1, ncls), lambda i: (0, 0)),
        ],
        out_specs=pl.BlockSpec((n, ncls), lambda i: (0, 0)),
        out_shape=jax.ShapeDtypeStruct((n, ncls), jnp.float32),
    )(x, W1, b1r, W2, b2r)
